# baseline (speedup 1.0000x reference)
import jax
import jax.numpy as jnp
from jax import lax
from jax.experimental import pallas as pl
from jax.experimental.pallas import tpu as pltpu
from jax.experimental.pallas import tpu_sc as plsc

B = 16384
F = 100
CARD = 20
OUT_D = 11
NPAIRS = F // 2
KP = 64
GATH = 56
PTROWS = NPAIRS * CARD * CARD
PD = 2 * OUT_D + 2
ROW_W = F * OUT_D

NC = 2
NS = 16
NW = NC * NS
ROWS_W = B // NW
NBUF = 16
NOUT = NBUF // 2
STEPS = ROWS_W // NBUF


def _sc_body(in_hbm, tab_hbm, out_hbm, *sc):
    in_v = sc[0]
    idx_vs = sc[1:1 + NBUF]
    dst_vs = sc[1 + NBUF:1 + 2 * NBUF]
    row_vs = sc[1 + 2 * NBUF:1 + 2 * NBUF + NOUT]
    tab_sh = sc[1 + 2 * NBUF + NOUT]
    gsem, osem = sc[-2], sc[-1]

    sid = lax.axis_index("s")
    wid = sid * NC + lax.axis_index("c")
    base = wid * ROWS_W

    @pl.when(sid == 0)
    def _():
        pltpu.sync_copy(tab_hbm, tab_sh)
    plsc.subcore_barrier()

    pltpu.sync_copy(in_hbm.at[pl.ds(base, ROWS_W)], in_v)

    lane = lax.broadcasted_iota(jnp.int32, (16,), 0)
    zeros = jnp.zeros((16,), jnp.int32)
    ngrp = KP // 16
    valid_last = lane < (NPAIRS - (ngrp - 1) * 16)
    idx_e, idx_o, pair_base = [], [], []
    for g in range(ngrp):
        k = lane + g * 16
        ok = k < NPAIRS
        idx_e.append(jnp.where(ok, 2 * k, zeros))
        idx_o.append(jnp.where(ok, 2 * k + 1, zeros))
        pair_base.append(k * (CARD * CARD))

    def step_fn(s, carry):
        @pl.when(s > 0)
        def _():
            for p in range(NOUT):
                for half in range(2):
                    pltpu.make_async_copy(out_hbm.at[0],
                                          row_vs[p].at[half], osem).wait()

        handles = []
        for j in range(NBUF):
            r = s * NBUF + j
            rv = jnp.full((16,), r, jnp.int32)
            ib = idx_vs[j]
            for g in range(ngrp):
                ca = plsc.load_gather(in_v, [rv, idx_e[g]]).astype(jnp.int32)
                cb = plsc.load_gather(in_v, [rv, idx_o[g]]).astype(jnp.int32)
                ci = ca * CARD + cb + pair_base[g]
                if g == ngrp - 1:
                    ci = jnp.where(valid_last, ci, zeros)
                ib[pl.ds(g * 16, 16)] = ci
            handles.append(
                pltpu.async_copy(tab_sh.at[ib.at[pl.ds(0, GATH)]],
                                 dst_vs[j], gsem))
        for p in range(NOUT):
            handles[2 * p].wait()
            handles[2 * p + 1].wait()
            rb = row_vs[p]
            for half in range(2):
                d = dst_vs[2 * p + half]
                for k in range(NPAIRS - 1):
                    rb[half, pl.ds(22 * k, 16)] = d[k, pl.ds(0, 16)]
                    rb[half, pl.ds(22 * k + 8, 16)] = d[k, pl.ds(8, 16)]
                k = NPAIRS - 1
                rb[half, pl.ds(22 * k, 16)] = d[k, pl.ds(0, 16)]
                rb[half, pl.ds(22 * k + 6, 16)] = d[k, pl.ds(6, 16)]
            gr = base + s * NBUF + 2 * p
            pltpu.async_copy(rb.at[0], out_hbm.at[gr], osem)
            pltpu.async_copy(rb.at[1], out_hbm.at[gr + 1], osem)
        return carry

    lax.fori_loop(0, STEPS, step_fn, 0)
    for p in range(NOUT):
        for half in range(2):
            pltpu.make_async_copy(out_hbm.at[0], row_vs[p].at[half],
                                  osem).wait()


def kernel(inputs, tables):
    tE = jnp.broadcast_to(tables[0::2][:, :, None, :],
                          (NPAIRS, CARD, CARD, OUT_D))
    tO = jnp.broadcast_to(tables[1::2][:, None, :, :],
                          (NPAIRS, CARD, CARD, OUT_D))
    pz = jnp.zeros((NPAIRS, CARD, CARD, PD - 2 * OUT_D), jnp.float32)
    ptab = jnp.concatenate([tE, tO, pz], axis=-1).reshape(PTROWS, PD)

    mesh = plsc.VectorSubcoreMesh(
        core_axis_name="c", subcore_axis_name="s",
        num_cores=NC, num_subcores=NS)
    run = pl.kernel(
        _sc_body,
        out_type=jax.ShapeDtypeStruct((B, ROW_W), jnp.float32),
        mesh=mesh,
        scratch_types=(
            [pltpu.VMEM((ROWS_W, F), jnp.float32)]
            + [pltpu.VMEM((KP,), jnp.int32) for _ in range(NBUF)]
            + [pltpu.VMEM((GATH, PD), jnp.float32) for _ in range(NBUF)]
            + [pltpu.VMEM((2, ROW_W), jnp.float32) for _ in range(NOUT)]
            + [pltpu.VMEM_SHARED((PTROWS, PD), jnp.float32)]
            + [pltpu.SemaphoreType.DMA, pltpu.SemaphoreType.DMA]
        ),
        compiler_params=pltpu.CompilerParams(use_tc_tiling_on_sc=False,
                                             needs_layout_passes=False),
    )
    return run(inputs, ptab)


# speedup vs baseline: 1.0147x; 1.0147x over previous
import jax
import jax.numpy as jnp
from jax import lax
from jax.experimental import pallas as pl
from jax.experimental.pallas import tpu as pltpu
from jax.experimental.pallas import tpu_sc as plsc

B = 16384
F = 100
CARD = 20
OUT_D = 11
NPAIRS = F // 2
KP = 64
GATH = 56
PTROWS = NPAIRS * CARD * CARD
PD = 2 * OUT_D + 2
ROW_W = F * OUT_D

NC = 2
NS = 16
NW = NC * NS
ROWS_W = B // NW
NBUF = 8
NOUT = NBUF // 2
STEPS = ROWS_W // NBUF


def _sc_body(in_hbm, tab_hbm, out_hbm, *sc):
    in_v = sc[0]
    idx_vs = sc[1:1 + NBUF]
    dst_vs = sc[1 + NBUF:1 + 2 * NBUF]
    row_vs = sc[1 + 2 * NBUF:1 + 2 * NBUF + NOUT]
    tab_sh = sc[1 + 2 * NBUF + NOUT]
    gsem, osem = sc[-2], sc[-1]

    sid = lax.axis_index("s")
    wid = sid * NC + lax.axis_index("c")
    base = wid * ROWS_W

    @pl.when(sid == 0)
    def _():
        pltpu.sync_copy(tab_hbm, tab_sh)
    plsc.subcore_barrier()

    pltpu.sync_copy(in_hbm.at[pl.ds(base, ROWS_W)], in_v)

    lane = lax.broadcasted_iota(jnp.int32, (16,), 0)
    zeros = jnp.zeros((16,), jnp.int32)
    ngrp = KP // 16
    valid_last = lane < (NPAIRS - (ngrp - 1) * 16)
    idx_e, idx_o, pair_base = [], [], []
    for g in range(ngrp):
        k = lane + g * 16
        ok = k < NPAIRS
        idx_e.append(jnp.where(ok, 2 * k, zeros))
        idx_o.append(jnp.where(ok, 2 * k + 1, zeros))
        pair_base.append(k * (CARD * CARD))

    def step_fn(s, carry):
        @pl.when(s > 0)
        def _():
            for p in range(NOUT):
                for half in range(2):
                    pltpu.make_async_copy(out_hbm.at[0],
                                          row_vs[p].at[half], osem).wait()

        handles = []
        for j in range(NBUF):
            r = s * NBUF + j
            rv = jnp.full((16,), r, jnp.int32)
            ib = idx_vs[j]
            for g in range(ngrp):
                ca = plsc.load_gather(in_v, [rv, idx_e[g]]).astype(jnp.int32)
                cb = plsc.load_gather(in_v, [rv, idx_o[g]]).astype(jnp.int32)
                ci = ca * CARD + cb + pair_base[g]
                if g == ngrp - 1:
                    ci = jnp.where(valid_last, ci, zeros)
                ib[pl.ds(g * 16, 16)] = ci
            handles.append(
                pltpu.async_copy(tab_sh.at[ib.at[pl.ds(0, GATH)]],
                                 dst_vs[j], gsem))
        for p in range(NOUT):
            handles[2 * p].wait()
            handles[2 * p + 1].wait()
            rb = row_vs[p]
            for half in range(2):
                d = dst_vs[2 * p + half]
                for k in range(NPAIRS - 1):
                    rb[half, pl.ds(22 * k, 16)] = d[k, pl.ds(0, 16)]
                    rb[half, pl.ds(22 * k + 8, 16)] = d[k, pl.ds(8, 16)]
                k = NPAIRS - 1
                rb[half, pl.ds(22 * k, 16)] = d[k, pl.ds(0, 16)]
                rb[half, pl.ds(22 * k + 6, 16)] = d[k, pl.ds(6, 16)]
            gr = base + s * NBUF + 2 * p
            pltpu.async_copy(rb.at[0], out_hbm.at[gr], osem)
            pltpu.async_copy(rb.at[1], out_hbm.at[gr + 1], osem)
        return carry

    lax.fori_loop(0, STEPS, step_fn, 0)
    for p in range(NOUT):
        for half in range(2):
            pltpu.make_async_copy(out_hbm.at[0], row_vs[p].at[half],
                                  osem).wait()


def kernel(inputs, tables):
    tEp = jnp.pad(tables[0::2], ((0, 0), (0, 0), (0, PD - OUT_D)))
    tOp = jnp.pad(tables[1::2], ((0, 0), (0, 0), (OUT_D, PD - 2 * OUT_D)))
    ptab = (jnp.repeat(tEp.reshape(NPAIRS * CARD, PD), CARD, axis=0)
            + jnp.tile(tOp, (1, CARD, 1)).reshape(PTROWS, PD))

    mesh = plsc.VectorSubcoreMesh(
        core_axis_name="c", subcore_axis_name="s",
        num_cores=NC, num_subcores=NS)
    run = pl.kernel(
        _sc_body,
        out_type=jax.ShapeDtypeStruct((B, ROW_W), jnp.float32),
        mesh=mesh,
        scratch_types=(
            [pltpu.VMEM((ROWS_W, F), jnp.float32)]
            + [pltpu.VMEM((KP,), jnp.int32) for _ in range(NBUF)]
            + [pltpu.VMEM((GATH, PD), jnp.float32) for _ in range(NBUF)]
            + [pltpu.VMEM((2, ROW_W), jnp.float32) for _ in range(NOUT)]
            + [pltpu.VMEM_SHARED((PTROWS, PD), jnp.float32)]
            + [pltpu.SemaphoreType.DMA, pltpu.SemaphoreType.DMA]
        ),
        compiler_params=pltpu.CompilerParams(use_tc_tiling_on_sc=False,
                                             needs_layout_passes=False),
    )
    return run(inputs, ptab)


# speedup vs baseline: 1.1237x; 1.1074x over previous
import jax
import jax.numpy as jnp
from jax import lax
from jax.experimental import pallas as pl
from jax.experimental.pallas import tpu as pltpu
from jax.experimental.pallas import tpu_sc as plsc

B = 16384
F = 100
CARD = 20
OUT_D = 11
NPAIRS = F // 2
KP = 64
GATH = 56
PTROWS = NPAIRS * CARD * CARD
PD = 2 * OUT_D + 2
ROW_W = F * OUT_D

NC = 2
NS = 16
NW = NC * NS
ROWS_W = B // NW
NBUF = 4
NOUT = NBUF // 2
STEPS = ROWS_W // NBUF


def _sc_body(in_hbm, tab_hbm, out_hbm, *sc):
    in_v = sc[0]
    idx_vs = sc[1:1 + NBUF]
    dst_vs = sc[1 + NBUF:1 + 2 * NBUF]
    row_vs = sc[1 + 2 * NBUF:1 + 2 * NBUF + NOUT]
    tab_sh = sc[1 + 2 * NBUF + NOUT]
    gsem, osem = sc[-2], sc[-1]

    sid = lax.axis_index("s")
    wid = sid * NC + lax.axis_index("c")
    base = wid * ROWS_W

    @pl.when(sid == 0)
    def _():
        pltpu.sync_copy(tab_hbm, tab_sh)
    plsc.subcore_barrier()

    pltpu.sync_copy(in_hbm.at[pl.ds(base, ROWS_W)], in_v)

    lane = lax.broadcasted_iota(jnp.int32, (16,), 0)
    zeros = jnp.zeros((16,), jnp.int32)
    ngrp = KP // 16
    valid_last = lane < (NPAIRS - (ngrp - 1) * 16)
    idx_e, idx_o, pair_base = [], [], []
    for g in range(ngrp):
        k = lane + g * 16
        ok = k < NPAIRS
        idx_e.append(jnp.where(ok, 2 * k, zeros))
        idx_o.append(jnp.where(ok, 2 * k + 1, zeros))
        pair_base.append(k * (CARD * CARD))

    def step_fn(s, carry):
        @pl.when(s > 0)
        def _():
            for p in range(NOUT):
                for half in range(2):
                    pltpu.make_async_copy(out_hbm.at[0],
                                          row_vs[p].at[half], osem).wait()

        handles = []
        for j in range(NBUF):
            r = s * NBUF + j
            rv = jnp.full((16,), r, jnp.int32)
            ib = idx_vs[j]
            for g in range(ngrp):
                ca = plsc.load_gather(in_v, [rv, idx_e[g]]).astype(jnp.int32)
                cb = plsc.load_gather(in_v, [rv, idx_o[g]]).astype(jnp.int32)
                ci = ca * CARD + cb + pair_base[g]
                if g == ngrp - 1:
                    ci = jnp.where(valid_last, ci, zeros)
                ib[pl.ds(g * 16, 16)] = ci
            handles.append(
                pltpu.async_copy(tab_sh.at[ib.at[pl.ds(0, GATH)]],
                                 dst_vs[j], gsem))
        for p in range(NOUT):
            handles[2 * p].wait()
            handles[2 * p + 1].wait()
            rb = row_vs[p]
            for half in range(2):
                d = dst_vs[2 * p + half]
                for k in range(NPAIRS - 1):
                    rb[half, pl.ds(22 * k, 16)] = d[k, pl.ds(0, 16)]
                    rb[half, pl.ds(22 * k + 8, 16)] = d[k, pl.ds(8, 16)]
                k = NPAIRS - 1
                rb[half, pl.ds(22 * k, 16)] = d[k, pl.ds(0, 16)]
                rb[half, pl.ds(22 * k + 6, 16)] = d[k, pl.ds(6, 16)]
            gr = base + s * NBUF + 2 * p
            pltpu.async_copy(rb.at[0], out_hbm.at[gr], osem)
            pltpu.async_copy(rb.at[1], out_hbm.at[gr + 1], osem)
        return carry

    lax.fori_loop(0, STEPS, step_fn, 0)
    for p in range(NOUT):
        for half in range(2):
            pltpu.make_async_copy(out_hbm.at[0], row_vs[p].at[half],
                                  osem).wait()


def kernel(inputs, tables):
    tE = jnp.broadcast_to(tables[0::2][:, :, None, :],
                          (NPAIRS, CARD, CARD, OUT_D))
    tO = jnp.broadcast_to(tables[1::2][:, None, :, :],
                          (NPAIRS, CARD, CARD, OUT_D))
    pz = jnp.zeros((NPAIRS, CARD, CARD, PD - 2 * OUT_D), jnp.float32)
    ptab = jnp.concatenate([tE, tO, pz], axis=-1).reshape(PTROWS, PD)

    mesh = plsc.VectorSubcoreMesh(
        core_axis_name="c", subcore_axis_name="s",
        num_cores=NC, num_subcores=NS)
    run = pl.kernel(
        _sc_body,
        out_type=jax.ShapeDtypeStruct((B, ROW_W), jnp.float32),
        mesh=mesh,
        scratch_types=(
            [pltpu.VMEM((ROWS_W, F), jnp.float32)]
            + [pltpu.VMEM((KP,), jnp.int32) for _ in range(NBUF)]
            + [pltpu.VMEM((GATH, PD), jnp.float32) for _ in range(NBUF)]
            + [pltpu.VMEM((2, ROW_W), jnp.float32) for _ in range(NOUT)]
            + [pltpu.VMEM_SHARED((PTROWS, PD), jnp.float32)]
            + [pltpu.SemaphoreType.DMA, pltpu.SemaphoreType.DMA]
        ),
        compiler_params=pltpu.CompilerParams(use_tc_tiling_on_sc=False,
                                             needs_layout_passes=False),
    )
    return run(inputs, ptab)


# speedup vs baseline: 1.1682x; 1.0397x over previous
import jax
import jax.numpy as jnp
from jax import lax
from jax.experimental import pallas as pl
from jax.experimental.pallas import tpu as pltpu
from jax.experimental.pallas import tpu_sc as plsc

B = 16384
F = 100
CARD = 20
OUT_D = 11
NPAIRS = F // 2
KP = 64
GATH = 56
PTROWS = NPAIRS * CARD * CARD
PD = 2 * OUT_D + 2
ROW_W = F * OUT_D
FIN = 128

NC = 2
NS = 16
NW = NC * NS
ROWS_W = B // NW
NBUF = 4
NOUT = NBUF // 2
STEPS = ROWS_W // NBUF


def _sc_body(in_hbm, tab_hbm, out_hbm, *sc):
    in_v = sc[0]
    idx_vs = sc[1:1 + NBUF]
    dst_vs = sc[1 + NBUF:1 + 2 * NBUF]
    row_vs = sc[1 + 2 * NBUF:1 + 2 * NBUF + NOUT]
    tab_sh = sc[1 + 2 * NBUF + NOUT]
    gsem, osem = sc[-2], sc[-1]

    sid = lax.axis_index("s")
    wid = sid * NC + lax.axis_index("c")
    base = wid * ROWS_W

    @pl.when(sid == 0)
    def _():
        pltpu.sync_copy(tab_hbm, tab_sh)
    plsc.subcore_barrier()

    pltpu.sync_copy(in_hbm.at[pl.ds(base, ROWS_W)], in_v)

    lane = lax.broadcasted_iota(jnp.int32, (16,), 0)
    zeros = jnp.zeros((16,), jnp.int32)
    ngrp = KP // 16
    valid_last = lane < (NPAIRS - (ngrp - 1) * 16)
    idx_e, idx_o, pair_base = [], [], []
    for g in range(ngrp):
        k = lane + g * 16
        ok = k < NPAIRS
        idx_e.append(jnp.where(ok, 2 * k, zeros))
        idx_o.append(jnp.where(ok, 2 * k + 1, zeros))
        pair_base.append(k * (CARD * CARD))

    def step_fn(s, carry):
        @pl.when(s > 0)
        def _():
            for p in range(NOUT):
                for half in range(2):
                    pltpu.make_async_copy(out_hbm.at[0],
                                          row_vs[p].at[half], osem).wait()

        handles = []
        for j in range(NBUF):
            r = s * NBUF + j
            rv = jnp.full((16,), r, jnp.int32)
            ib = idx_vs[j]
            for g in range(ngrp):
                ca = plsc.load_gather(in_v, [rv, idx_e[g]]).astype(jnp.int32)
                cb = plsc.load_gather(in_v, [rv, idx_o[g]]).astype(jnp.int32)
                ci = ca * CARD + cb + pair_base[g]
                if g == ngrp - 1:
                    ci = jnp.where(valid_last, ci, zeros)
                ib[pl.ds(g * 16, 16)] = ci
            handles.append(
                pltpu.async_copy(tab_sh.at[ib.at[pl.ds(0, GATH)]],
                                 dst_vs[j], gsem))
        for p in range(NOUT):
            handles[2 * p].wait()
            handles[2 * p + 1].wait()
            rb = row_vs[p]
            for half in range(2):
                d = dst_vs[2 * p + half]
                for k in range(NPAIRS - 1):
                    rb[half, pl.ds(22 * k, 16)] = d[k, pl.ds(0, 16)]
                    rb[half, pl.ds(22 * k + 8, 16)] = d[k, pl.ds(8, 16)]
                k = NPAIRS - 1
                rb[half, pl.ds(22 * k, 16)] = d[k, pl.ds(0, 16)]
                rb[half, pl.ds(22 * k + 6, 16)] = d[k, pl.ds(6, 16)]
            gr = base + s * NBUF + 2 * p
            pltpu.async_copy(rb.at[0], out_hbm.at[gr], osem)
            pltpu.async_copy(rb.at[1], out_hbm.at[gr + 1], osem)
        return carry

    lax.fori_loop(0, STEPS, step_fn, 0)
    for p in range(NOUT):
        for half in range(2):
            pltpu.make_async_copy(out_hbm.at[0], row_vs[p].at[half],
                                  osem).wait()


def kernel(inputs, tables):
    tE = jnp.broadcast_to(tables[0::2][:, :, None, :],
                          (NPAIRS, CARD, CARD, OUT_D))
    tO = jnp.broadcast_to(tables[1::2][:, None, :, :],
                          (NPAIRS, CARD, CARD, OUT_D))
    pz = jnp.zeros((NPAIRS, CARD, CARD, PD - 2 * OUT_D), jnp.float32)
    ptab = jnp.concatenate([tE, tO, pz], axis=-1).reshape(PTROWS, PD)

    mesh = plsc.VectorSubcoreMesh(
        core_axis_name="c", subcore_axis_name="s",
        num_cores=NC, num_subcores=NS)
    run = pl.kernel(
        _sc_body,
        out_type=jax.ShapeDtypeStruct((B, ROW_W), jnp.float32),
        mesh=mesh,
        scratch_types=(
            [pltpu.VMEM((ROWS_W, FIN), jnp.float32)]
            + [pltpu.VMEM((KP,), jnp.int32) for _ in range(NBUF)]
            + [pltpu.VMEM((GATH, PD), jnp.float32) for _ in range(NBUF)]
            + [pltpu.VMEM((2, ROW_W), jnp.float32) for _ in range(NOUT)]
            + [pltpu.VMEM_SHARED((PTROWS, PD), jnp.float32)]
            + [pltpu.SemaphoreType.DMA, pltpu.SemaphoreType.DMA]
        ),
        compiler_params=pltpu.CompilerParams(use_tc_tiling_on_sc=False,
                                             needs_layout_passes=False),
    )
    return run(jnp.pad(inputs, ((0, 0), (0, FIN - F))), ptab)
